# baseline (speedup 1.0000x reference)
import dataclasses
import functools

import jax
import jax.numpy as jnp
from jax import lax
from jax.experimental import pallas as pl
from jax.experimental.pallas import tpu as pltpu
from jax.experimental.pallas import tpu_sc as plsc

DZ = 0.5
NS = 64
L = 16
NB = 8
UNROLL = 8


def kernel(xyz1, xyz2):
    b, _, n = xyz1.shape
    m = xyz2.shape[2]
    bm = b * m
    nblocks = m // NB

    mesh = plsc.VectorSubcoreMesh(core_axis_name="c", subcore_axis_name="s")
    cp = pltpu.CompilerParams()
    if "needs_layout_passes" in pltpu.CompilerParams.__dataclass_fields__:
        cp = dataclasses.replace(cp, needs_layout_passes=False)
    if "use_tc_tiling_on_sc" in pltpu.CompilerParams.__dataclass_fields__:
        cp = dataclasses.replace(cp, use_tc_tiling_on_sc=True)

    @functools.partial(
        pl.kernel,
        mesh=mesh,
        compiler_params=cp,
        out_type=(
            jax.ShapeDtypeStruct((b, m, NS), jnp.int32),
            jax.ShapeDtypeStruct((b, m), jnp.int32),
        ),
        scratch_types=[
            pltpu.VMEM((1, n), jnp.float32),
            pltpu.VMEM((1, m), jnp.float32),
            pltpu.VMEM((NB, NS), jnp.int32),
            pltpu.VMEM((NB,), jnp.int32),
            pltpu.VMEM((NS + UNROLL * L,), jnp.int32),
            pltpu.SMEM((1,), jnp.int32),
        ],
    )
    def qdp(xyz1_hbm, xyz2_hbm, idx_hbm, cnt_hbm,
            z1_v, z2_v, out_v, cnt_v, buf_v, ctr_s):
        batch = lax.axis_index("c")
        sid = lax.axis_index("s")

        pltpu.sync_copy(xyz1_hbm.at[batch, pl.ds(2, 1), :], z1_v)
        pltpu.sync_copy(xyz2_hbm.at[batch, pl.ds(2, 1), :], z2_v)

        @pl.when(sid == 0)
        def _():
            ctr_s[0] = 0

        plsc.subcore_barrier()

        lanes = lax.iota(jnp.int32, L)
        zvec16 = jnp.zeros((L,), jnp.int32)

        def process_block(blk):
            @pl.loop(0, NB)
            def per_query(r):
                q = blk * NB + r
                z2q = plsc.load_gather(z2_v, [zvec16, zvec16 + q])

                def cond(carry):
                    k, cs = carry
                    return jnp.logical_and(k < n, cs < NS)

                def body(carry):
                    k, cs = carry
                    zs = [z1_v[0, pl.ds(k + u * L, L)] for u in range(UNROLL)]
                    masks = [jnp.abs(zc - z2q) < DZ for zc in zs]
                    pops = [plsc.all_reduce_population_count(mk)[0] for mk in masks]
                    offs = [cs]
                    for u in range(UNROLL):
                        offs.append(offs[u] + pops[u])
                    for u in range(UNROLL):
                        plsc.store_compressed(
                            buf_v.at[pl.ds(offs[u], L)], k + (u * L) + lanes,
                            mask=masks[u])
                    return k + UNROLL * L, offs[UNROLL]

                _, cs = lax.while_loop(cond, body, (jnp.int32(0), jnp.int32(0)))

                datas = [buf_v[pl.ds(j * L, L)] for j in range(NS // L)]
                first_s = datas[0][0]
                collv = jnp.full((L,), cs, jnp.int32)
                firstv = jnp.where(collv > 0, jnp.full((L,), first_s, jnp.int32), 0)
                for j in range(NS // L):
                    outv = jnp.where(lanes + (j * L) < collv, datas[j], firstv)
                    out_v[r, pl.ds(j * L, L)] = outv
                pts = jnp.full((L,), jnp.minimum(cs, NS), jnp.int32)
                plsc.store_scatter(cnt_v, [zvec16 + r], pts, mask=lanes == 0)

            qoff = blk * NB
            pltpu.sync_copy(out_v, idx_hbm.at[batch, pl.ds(qoff, NB), :])
            pltpu.sync_copy(cnt_v, cnt_hbm.at[batch, pl.ds(qoff, NB)])

        def wcond(blk):
            return blk < nblocks

        def wbody(blk):
            process_block(blk)
            return plsc.fetch_and_add(ctr_s.at[0], 1, subcore_id=0)

        blk0 = plsc.fetch_and_add(ctr_s.at[0], 1, subcore_id=0)
        lax.while_loop(wcond, wbody, blk0)

    idx3d, pts_cnt = qdp(xyz1, xyz2)
    return idx3d.astype(jnp.int64), pts_cnt


# speedup vs baseline: 1.0036x; 1.0036x over previous
import dataclasses
import functools

import jax
import jax.numpy as jnp
from jax import lax
from jax.experimental import pallas as pl
from jax.experimental.pallas import tpu as pltpu
from jax.experimental.pallas import tpu_sc as plsc

DZ = 0.5
NS = 64
L = 16
NB = 8
UNROLL = 8


def kernel(xyz1, xyz2):
    b, _, n = xyz1.shape
    m = xyz2.shape[2]
    bm = b * m
    nblocks = m // NB

    mesh = plsc.VectorSubcoreMesh(core_axis_name="c", subcore_axis_name="s")
    cp = pltpu.CompilerParams()
    if "needs_layout_passes" in pltpu.CompilerParams.__dataclass_fields__:
        cp = dataclasses.replace(cp, needs_layout_passes=False)

    @functools.partial(
        pl.kernel,
        mesh=mesh,
        compiler_params=cp,
        out_type=(
            jax.ShapeDtypeStruct((b, m, NS), jnp.int32),
            jax.ShapeDtypeStruct((b, m), jnp.int32),
        ),
        scratch_types=[
            pltpu.VMEM((1, n), jnp.float32),
            pltpu.VMEM((1, m), jnp.float32),
            pltpu.VMEM((NB, NS), jnp.int32),
            pltpu.VMEM((NB,), jnp.int32),
            pltpu.VMEM((NS + UNROLL * L,), jnp.int32),
            pltpu.SMEM((1,), jnp.int32),
        ],
    )
    def qdp(xyz1_hbm, xyz2_hbm, idx_hbm, cnt_hbm,
            z1_v, z2_v, out_v, cnt_v, buf_v, ctr_s):
        batch = lax.axis_index("c")
        sid = lax.axis_index("s")

        pltpu.sync_copy(xyz1_hbm.at[batch, pl.ds(2, 1), :], z1_v)
        pltpu.sync_copy(xyz2_hbm.at[batch, pl.ds(2, 1), :], z2_v)

        @pl.when(sid == 0)
        def _():
            ctr_s[0] = 0

        plsc.subcore_barrier()

        lanes = lax.iota(jnp.int32, L)
        zvec16 = jnp.zeros((L,), jnp.int32)

        def process_block(blk):
            @pl.loop(0, NB)
            def per_query(r):
                q = blk * NB + r
                z2q = plsc.load_gather(z2_v, [zvec16, zvec16 + q])

                def cond(carry):
                    k, cs = carry
                    return jnp.logical_and(k < n, cs < NS)

                def body(carry):
                    k, cs = carry
                    zs = [z1_v[0, pl.ds(k + u * L, L)] for u in range(UNROLL)]
                    masks = [jnp.abs(zc - z2q) < DZ for zc in zs]
                    pops = [plsc.all_reduce_population_count(mk)[0] for mk in masks]
                    offs = [cs]
                    for u in range(UNROLL):
                        offs.append(offs[u] + pops[u])
                    for u in range(UNROLL):
                        plsc.store_compressed(
                            buf_v.at[pl.ds(offs[u], L)], k + (u * L) + lanes,
                            mask=masks[u])
                    return k + UNROLL * L, offs[UNROLL]

                _, cs = lax.while_loop(cond, body, (jnp.int32(0), jnp.int32(0)))

                datas = [buf_v[pl.ds(j * L, L)] for j in range(NS // L)]
                first_s = datas[0][0]
                collv = jnp.full((L,), cs, jnp.int32)
                firstv = jnp.where(collv > 0, jnp.full((L,), first_s, jnp.int32), 0)
                for j in range(NS // L):
                    outv = jnp.where(lanes + (j * L) < collv, datas[j], firstv)
                    out_v[r, pl.ds(j * L, L)] = outv
                pts = jnp.full((L,), jnp.minimum(cs, NS), jnp.int32)
                plsc.store_scatter(cnt_v, [zvec16 + r], pts, mask=lanes == 0)

            qoff = blk * NB
            pltpu.sync_copy(out_v, idx_hbm.at[batch, pl.ds(qoff, NB), :])
            pltpu.sync_copy(cnt_v, cnt_hbm.at[batch, pl.ds(qoff, NB)])

        def wcond(blk):
            return blk < nblocks

        def wbody(blk):
            process_block(blk)
            return plsc.fetch_and_add(ctr_s.at[0], 1, subcore_id=0)

        blk0 = plsc.fetch_and_add(ctr_s.at[0], 1, subcore_id=0)
        lax.while_loop(wcond, wbody, blk0)

    idx3d, pts_cnt = qdp(xyz1, xyz2)
    return idx3d.astype(jnp.int64), pts_cnt
